# baseline (speedup 1.0000x reference)
import dataclasses
import functools

import jax
import jax.numpy as jnp
from jax import lax
from jax.experimental import pallas as pl
from jax.experimental.pallas import tpu as pltpu
from jax.experimental.pallas import tpu_sc as plsc

N = 10000
E = 320000
F_IN = 128
HID = 256

_C = 128
_EP = ((E + 8191) // 8192) * 8192
_NPAD = 10240

_mesh = plsc.VectorSubcoreMesh(core_axis_name="c", subcore_axis_name="s")

_cp = pltpu.CompilerParams()
if "needs_layout_passes" in pltpu.CompilerParams.__dataclass_fields__:
    _cp = dataclasses.replace(_cp, needs_layout_passes=False)


@functools.partial(
    pl.kernel,
    mesh=_mesh,
    out_type=jax.ShapeDtypeStruct((32, _NPAD), jnp.float32),
    compiler_params=_cp,
    scratch_types=[
        pltpu.VMEM((1, _C), jnp.int32),
        pltpu.VMEM((_NPAD,), jnp.float32),
        pltpu.SemaphoreType.DMA,
    ],
)
def _deg_kernel(dst_hbm, out_hbm, dst_v, dacc, sem):
    sid = lax.axis_index("s")
    cid = lax.axis_index("c")
    wid = sid * 2 + cid

    @pl.loop(0, _NPAD, step=16)
    def _(i):
        dacc[pl.ds(i, 16)] = jnp.zeros((16,), jnp.float32)

    rows_w = _EP // 32 // _C
    ones = jnp.ones((16,), jnp.float32)

    @pl.loop(0, rows_w)
    def _(j):
        pltpu.sync_copy(dst_hbm.at[wid * rows_w + j], dst_v.at[0])

        @pl.loop(0, _C, step=16)
        def _(i):
            idx = dst_v[0, pl.ds(i, 16)]
            plsc.addupdate_scatter(dacc, [idx], ones)

    pltpu.sync_copy(dacc, out_hbm.at[wid])


def _make_agg(split_edges):
    ept = (_EP // 32) if split_edges else (_EP // 16)
    tile_rows = ept // _C
    zr = _NPAD // 16
    KB = 8

    @functools.partial(
        pl.kernel,
        mesh=_mesh,
        out_type=jax.ShapeDtypeStruct((2, _NPAD, 128), jnp.float32),
        scratch_types=[
            pltpu.VMEM((KB, _C), jnp.int32),
            pltpu.VMEM((KB, _C), jnp.int32),
            pltpu.VMEM((_C, 128), jnp.float32),
            pltpu.VMEM((_C, 128), jnp.float32),
            pltpu.VMEM_SHARED((_NPAD, 128), jnp.float32),
            pltpu.SemaphoreType.DMA,
            pltpu.SemaphoreType.DMA,
        ],
    )
    def agg(g_hbm, src_hbm, dst_hbm, z_hbm, out_hbm,
            src_blk, dst_blk, rows0, rows1, acc, gsem, ssem):
        sid = lax.axis_index("s")
        cid = lax.axis_index("c")
        rows = (rows0, rows1)

        pltpu.sync_copy(z_hbm, acc.at[pl.ds(sid * zr, zr)])
        plsc.subcore_barrier()

        def run(gt, ot, row_off):
            @pl.loop(0, tile_rows // KB)
            def _(b):
                r0 = row_off + sid * tile_rows + b * KB
                pltpu.sync_copy(src_hbm.at[pl.ds(r0, KB)], src_blk)
                pltpu.sync_copy(dst_hbm.at[pl.ds(r0, KB)], dst_blk)
                pltpu.async_copy(gt.at[src_blk.at[0]], rows[0], gsem)
                for k in range(KB):
                    cur = rows[k % 2]
                    if k >= 1:
                        pltpu.make_async_copy(
                            rows[(k - 1) % 2],
                            acc.at[dst_blk.at[k - 1]], ssem).wait()
                    if k + 1 < KB:
                        pltpu.async_copy(gt.at[src_blk.at[k + 1]],
                                         rows[(k + 1) % 2], gsem)
                    pltpu.make_async_copy(gt.at[src_blk.at[k]],
                                          cur, gsem).wait()
                    pltpu.async_copy(cur, acc.at[dst_blk.at[k]], ssem,
                                     add=True)
                pltpu.make_async_copy(rows[(KB - 1) % 2],
                                      acc.at[dst_blk.at[KB - 1]],
                                      ssem).wait()

            plsc.subcore_barrier()
            pltpu.sync_copy(acc.at[pl.ds(sid * zr, zr)],
                            ot.at[pl.ds(sid * zr, zr)])

        if split_edges:
            @pl.when(cid == 0)
            def _():
                run(g_hbm, out_hbm.at[0], 0)

            @pl.when(cid == 1)
            def _():
                run(g_hbm, out_hbm.at[1], _EP // 256)
        else:
            @pl.when(cid == 0)
            def _():
                run(g_hbm.at[0], out_hbm.at[0], 0)

            @pl.when(cid == 1)
            def _():
                run(g_hbm.at[1], out_hbm.at[1], 0)

    return agg


_agg_edge = _make_agg(True)
_agg_feat = _make_agg(False)


def _dinv_body(dp_ref, dinv_ref):
    deg = jnp.sum(dp_ref[...], axis=0, keepdims=True) + 1.0
    dinv_ref[...] = lax.rsqrt(deg)


def _scale_body(x_ref, dinv_ref, o_ref):
    o_ref[...] = x_ref[...] * dinv_ref[...]


def _stage1_body(sa_ref, sb_ref, g_ref, dinv_ref, w_ref, b_ref, o_ref):
    h = (sa_ref[...] + sb_ref[...] + g_ref[...]) * dinv_ref[...]
    y = jnp.maximum(jnp.dot(h, w_ref[...],
                            preferred_element_type=jnp.float32)
                    + b_ref[...], 0.0)
    o_ref[...] = y * dinv_ref[...]


def _stage2_body(s_ref, g_ref, dinv_ref, w2_ref, b2_ref, w3_ref, o_ref):
    h = (s_ref[...] + g_ref[...]) * dinv_ref[...]
    y = jnp.maximum(jnp.dot(h, w2_ref[...],
                            preferred_element_type=jnp.float32)
                    + b2_ref[...], 0.0)
    z = jnp.dot(y, w3_ref[...], preferred_element_type=jnp.float32)
    o_ref[...] = z * dinv_ref[...]


def _stage3_body(sa_ref, sb_ref, g_ref, dinv_ref, b_ref, o_ref):
    o_ref[...] = ((sa_ref[...] + sb_ref[...] + g_ref[...]) * dinv_ref[...]
                  + b_ref[...])


_BN = 2000


def _row_spec(f):
    return pl.BlockSpec((_BN, f), lambda i: (i, 0))


def _full_spec(shape):
    return pl.BlockSpec(shape, lambda i: (0, 0))


def kernel(x, edge_index, W1, b1, W2, b2, W3, b3):
    src = edge_index[0]
    dst = edge_index[1]
    pad = _EP - E
    pad_i = jnp.arange(pad, dtype=jnp.int32)
    src_p = jnp.concatenate([src, pad_i % N]).reshape(_EP // _C, _C)
    dst_p = jnp.concatenate([dst, N + (pad_i % (_NPAD - N))]
                            ).reshape(_EP // _C, _C)

    zeros128 = jnp.zeros((_NPAD // 16, 128), jnp.float32)

    dp = _deg_kernel(dst_p)
    dinv_row = pl.pallas_call(
        _dinv_body,
        out_shape=jax.ShapeDtypeStruct((1, _NPAD), jnp.float32),
    )(dp)
    dinv = dinv_row.reshape(_NPAD, 1)[:N]

    g0 = pl.pallas_call(
        _scale_body,
        grid=(N // _BN,),
        in_specs=[_row_spec(F_IN), _row_spec(1)],
        out_specs=_row_spec(F_IN),
        out_shape=jax.ShapeDtypeStruct((N, F_IN), jnp.float32),
    )(x, dinv)
    s0 = _agg_edge(g0, src_p, dst_p, zeros128)

    g1 = pl.pallas_call(
        _stage1_body,
        grid=(N // _BN,),
        in_specs=[_row_spec(F_IN), _row_spec(F_IN), _row_spec(F_IN),
                  _row_spec(1), _full_spec((F_IN, HID)), _full_spec((1, HID))],
        out_specs=_row_spec(HID),
        out_shape=jax.ShapeDtypeStruct((N, HID), jnp.float32),
    )(s0[0, :N], s0[1, :N], g0, dinv, W1, b1.reshape(1, HID))

    g1s = g1.reshape(N, 2, 128).transpose(1, 0, 2)
    s1 = _agg_feat(g1s, src_p, dst_p, zeros128)[:, :N]
    s1f = s1.transpose(1, 0, 2).reshape(N, HID)

    g2 = pl.pallas_call(
        _stage2_body,
        grid=(N // _BN,),
        in_specs=[_row_spec(HID), _row_spec(HID), _row_spec(1),
                  _full_spec((HID, HID)), _full_spec((1, HID)),
                  _full_spec((HID, F_IN))],
        out_specs=_row_spec(F_IN),
        out_shape=jax.ShapeDtypeStruct((N, F_IN), jnp.float32),
    )(s1f, g1, dinv, W2, b2.reshape(1, HID), W3)

    s2 = _agg_edge(g2, src_p, dst_p, zeros128)

    out = pl.pallas_call(
        _stage3_body,
        grid=(N // _BN,),
        in_specs=[_row_spec(F_IN), _row_spec(F_IN), _row_spec(F_IN),
                  _row_spec(1), _full_spec((1, F_IN))],
        out_specs=_row_spec(F_IN),
        out_shape=jax.ShapeDtypeStruct((N, F_IN), jnp.float32),
    )(s2[0, :N], s2[1, :N], g2, dinv, b3.reshape(1, F_IN))
    return out


# speedup vs baseline: 1.1161x; 1.1161x over previous
import dataclasses
import functools

import jax
import jax.numpy as jnp
from jax import lax
from jax.experimental import pallas as pl
from jax.experimental.pallas import tpu as pltpu
from jax.experimental.pallas import tpu_sc as plsc

N = 10000
E = 320000
F_IN = 128
HID = 256

_C = 128
_EP = ((E + 8191) // 8192) * 8192
_NPAD = 10240

_mesh = plsc.VectorSubcoreMesh(core_axis_name="c", subcore_axis_name="s")

_cp = pltpu.CompilerParams()
if "needs_layout_passes" in pltpu.CompilerParams.__dataclass_fields__:
    _cp = dataclasses.replace(_cp, needs_layout_passes=False)


@functools.partial(
    pl.kernel,
    mesh=_mesh,
    out_type=jax.ShapeDtypeStruct((32, _NPAD), jnp.float32),
    compiler_params=_cp,
    scratch_types=[
        pltpu.VMEM((_EP // 32 // _C, _C), jnp.int32),
        pltpu.VMEM((_NPAD,), jnp.float32),
        pltpu.SemaphoreType.DMA,
    ],
)
def _deg_kernel(dst_hbm, out_hbm, dst_v, dacc, sem):
    sid = lax.axis_index("s")
    cid = lax.axis_index("c")
    wid = sid * 2 + cid

    @pl.loop(0, _NPAD, step=16)
    def _(i):
        dacc[pl.ds(i, 16)] = jnp.zeros((16,), jnp.float32)

    rows_w = _EP // 32 // _C
    ones = jnp.ones((16,), jnp.float32)
    pltpu.sync_copy(dst_hbm.at[pl.ds(wid * rows_w, rows_w)], dst_v)

    @pl.loop(0, rows_w)
    def _(j):
        @pl.loop(0, _C, step=16)
        def _(i):
            idx = dst_v[j, pl.ds(i, 16)]
            plsc.addupdate_scatter(dacc, [idx], ones)

    pltpu.sync_copy(dacc, out_hbm.at[wid])


def _make_agg(split_edges):
    ept = (_EP // 32) if split_edges else (_EP // 16)
    tile_rows = ept // _C
    zr = _NPAD // 16
    KB = 16

    @functools.partial(
        pl.kernel,
        mesh=_mesh,
        out_type=jax.ShapeDtypeStruct((2, _NPAD, 128), jnp.float32),
        scratch_types=[
            pltpu.VMEM((KB, _C), jnp.int32),
            pltpu.VMEM((KB, _C), jnp.int32),
            pltpu.VMEM((_C, 128), jnp.float32),
            pltpu.VMEM((_C, 128), jnp.float32),
            pltpu.VMEM_SHARED((_NPAD, 128), jnp.float32),
            pltpu.SemaphoreType.DMA,
            pltpu.SemaphoreType.DMA,
        ],
    )
    def agg(g_hbm, src_hbm, dst_hbm, z_hbm, out_hbm,
            src_blk, dst_blk, rows0, rows1, acc, gsem, ssem):
        sid = lax.axis_index("s")
        cid = lax.axis_index("c")
        rows = (rows0, rows1)

        pltpu.sync_copy(z_hbm, acc.at[pl.ds(sid * zr, zr)])
        plsc.subcore_barrier()

        def run(gt, ot, row_off):
            @pl.loop(0, tile_rows // KB)
            def _(b):
                r0 = row_off + sid * tile_rows + b * KB
                pltpu.sync_copy(src_hbm.at[pl.ds(r0, KB)], src_blk)
                pltpu.sync_copy(dst_hbm.at[pl.ds(r0, KB)], dst_blk)
                pltpu.async_copy(gt.at[src_blk.at[0]], rows[0], gsem)
                for k in range(KB):
                    cur = rows[k % 2]
                    if k >= 1:
                        pltpu.make_async_copy(
                            rows[(k - 1) % 2],
                            acc.at[dst_blk.at[k - 1]], ssem).wait()
                    if k + 1 < KB:
                        pltpu.async_copy(gt.at[src_blk.at[k + 1]],
                                         rows[(k + 1) % 2], gsem)
                    pltpu.make_async_copy(gt.at[src_blk.at[k]],
                                          cur, gsem).wait()
                    pltpu.async_copy(cur, acc.at[dst_blk.at[k]], ssem,
                                     add=True)
                pltpu.make_async_copy(rows[(KB - 1) % 2],
                                      acc.at[dst_blk.at[KB - 1]],
                                      ssem).wait()

            plsc.subcore_barrier()
            pltpu.sync_copy(acc.at[pl.ds(sid * zr, zr)],
                            ot.at[pl.ds(sid * zr, zr)])

        if split_edges:
            @pl.when(cid == 0)
            def _():
                run(g_hbm, out_hbm.at[0], 0)

            @pl.when(cid == 1)
            def _():
                run(g_hbm, out_hbm.at[1], _EP // 256)
        else:
            @pl.when(cid == 0)
            def _():
                run(g_hbm.at[0], out_hbm.at[0], 0)

            @pl.when(cid == 1)
            def _():
                run(g_hbm.at[1], out_hbm.at[1], 0)

    return agg


_agg_edge = _make_agg(True)
_agg_feat = _make_agg(False)


def _dinv_body(dp_ref, dinv_ref):
    deg = jnp.sum(dp_ref[...], axis=0, keepdims=True) + 1.0
    dinv_ref[...] = lax.rsqrt(deg)


def _scale_body(x_ref, dinv_ref, o_ref):
    o_ref[...] = x_ref[...] * dinv_ref[...]


def _stage1_body(sa_ref, sb_ref, g_ref, dinv_ref, w_ref, b_ref, o_ref):
    h = (sa_ref[...] + sb_ref[...] + g_ref[...]) * dinv_ref[...]
    y = jnp.maximum(jnp.dot(h, w_ref[...],
                            preferred_element_type=jnp.float32)
                    + b_ref[...], 0.0)
    o_ref[...] = y * dinv_ref[...]


def _stage2_body(s_ref, g_ref, dinv_ref, w2_ref, b2_ref, w3_ref, o_ref):
    h = (s_ref[...] + g_ref[...]) * dinv_ref[...]
    y = jnp.maximum(jnp.dot(h, w2_ref[...],
                            preferred_element_type=jnp.float32)
                    + b2_ref[...], 0.0)
    z = jnp.dot(y, w3_ref[...], preferred_element_type=jnp.float32)
    o_ref[...] = z * dinv_ref[...]


def _stage3_body(sa_ref, sb_ref, g_ref, dinv_ref, b_ref, o_ref):
    o_ref[...] = ((sa_ref[...] + sb_ref[...] + g_ref[...]) * dinv_ref[...]
                  + b_ref[...])


_BN = 2000


def _row_spec(f):
    return pl.BlockSpec((_BN, f), lambda i: (i, 0))


def _full_spec(shape):
    return pl.BlockSpec(shape, lambda i: (0, 0))


def kernel(x, edge_index, W1, b1, W2, b2, W3, b3):
    src = edge_index[0]
    dst = edge_index[1]
    pad = _EP - E
    pad_i = jnp.arange(pad, dtype=jnp.int32)
    src_p = jnp.concatenate([src, pad_i % N]).reshape(_EP // _C, _C)
    dst_p = jnp.concatenate([dst, N + (pad_i % (_NPAD - N))]
                            ).reshape(_EP // _C, _C)

    zeros128 = jnp.zeros((_NPAD // 16, 128), jnp.float32)

    dp = _deg_kernel(dst_p)
    dinv_row = pl.pallas_call(
        _dinv_body,
        out_shape=jax.ShapeDtypeStruct((1, _NPAD), jnp.float32),
    )(dp)
    dinv = dinv_row.reshape(_NPAD, 1)[:N]

    g0 = pl.pallas_call(
        _scale_body,
        grid=(N // _BN,),
        in_specs=[_row_spec(F_IN), _row_spec(1)],
        out_specs=_row_spec(F_IN),
        out_shape=jax.ShapeDtypeStruct((N, F_IN), jnp.float32),
    )(x, dinv)
    s0 = _agg_edge(g0, src_p, dst_p, zeros128)

    g1 = pl.pallas_call(
        _stage1_body,
        grid=(N // _BN,),
        in_specs=[_row_spec(F_IN), _row_spec(F_IN), _row_spec(F_IN),
                  _row_spec(1), _full_spec((F_IN, HID)), _full_spec((1, HID))],
        out_specs=_row_spec(HID),
        out_shape=jax.ShapeDtypeStruct((N, HID), jnp.float32),
    )(s0[0, :N], s0[1, :N], g0, dinv, W1, b1.reshape(1, HID))

    g1s = g1.reshape(N, 2, 128).transpose(1, 0, 2)
    s1 = _agg_feat(g1s, src_p, dst_p, zeros128)[:, :N]
    s1f = s1.transpose(1, 0, 2).reshape(N, HID)

    g2 = pl.pallas_call(
        _stage2_body,
        grid=(N // _BN,),
        in_specs=[_row_spec(HID), _row_spec(HID), _row_spec(1),
                  _full_spec((HID, HID)), _full_spec((1, HID)),
                  _full_spec((HID, F_IN))],
        out_specs=_row_spec(F_IN),
        out_shape=jax.ShapeDtypeStruct((N, F_IN), jnp.float32),
    )(s1f, g1, dinv, W2, b2.reshape(1, HID), W3)

    s2 = _agg_edge(g2, src_p, dst_p, zeros128)

    out = pl.pallas_call(
        _stage3_body,
        grid=(N // _BN,),
        in_specs=[_row_spec(F_IN), _row_spec(F_IN), _row_spec(F_IN),
                  _row_spec(1), _full_spec((1, F_IN))],
        out_specs=_row_spec(F_IN),
        out_shape=jax.ShapeDtypeStruct((N, F_IN), jnp.float32),
    )(s2[0, :N], s2[1, :N], g2, dinv, b3.reshape(1, F_IN))
    return out


# speedup vs baseline: 1.2078x; 1.0822x over previous
import dataclasses
import functools

import jax
import jax.numpy as jnp
from jax import lax
from jax.experimental import pallas as pl
from jax.experimental.pallas import tpu as pltpu
from jax.experimental.pallas import tpu_sc as plsc

N = 10000
E = 320000
F_IN = 128
HID = 256

_C = 128
_EP = ((E + 8191) // 8192) * 8192
_NPAD = 10240

_mesh = plsc.VectorSubcoreMesh(core_axis_name="c", subcore_axis_name="s")

_cp = pltpu.CompilerParams()
if "needs_layout_passes" in pltpu.CompilerParams.__dataclass_fields__:
    _cp = dataclasses.replace(_cp, needs_layout_passes=False)


@functools.partial(
    pl.kernel,
    mesh=_mesh,
    out_type=jax.ShapeDtypeStruct((32, _NPAD), jnp.float32),
    compiler_params=_cp,
    scratch_types=[
        pltpu.VMEM((_EP // 32 // _C, _C), jnp.int32),
        pltpu.VMEM((_NPAD,), jnp.float32),
        pltpu.SemaphoreType.DMA,
    ],
)
def _deg_kernel(dst_hbm, out_hbm, dst_v, dacc, sem):
    sid = lax.axis_index("s")
    cid = lax.axis_index("c")
    wid = sid * 2 + cid

    @pl.loop(0, _NPAD, step=16)
    def _(i):
        dacc[pl.ds(i, 16)] = jnp.zeros((16,), jnp.float32)

    rows_w = _EP // 32 // _C
    ones = jnp.ones((16,), jnp.float32)
    pltpu.sync_copy(dst_hbm.at[pl.ds(wid * rows_w, rows_w)], dst_v)

    @pl.loop(0, rows_w)
    def _(j):
        @pl.loop(0, _C, step=16)
        def _(i):
            idx = dst_v[j, pl.ds(i, 16)]
            plsc.addupdate_scatter(dacc, [idx], ones)

    pltpu.sync_copy(dacc, out_hbm.at[wid])


def _make_agg(split_edges):
    ept = (_EP // 32) if split_edges else (_EP // 16)
    tile_rows = ept // _C
    zr = _NPAD // 16
    KB = 16

    @functools.partial(
        pl.kernel,
        mesh=_mesh,
        out_type=jax.ShapeDtypeStruct((2, _NPAD, 128), jnp.float32),
        scratch_types=[
            pltpu.VMEM((KB, _C), jnp.int32),
            pltpu.VMEM((KB, _C), jnp.int32),
            pltpu.VMEM((_C, 128), jnp.float32),
            pltpu.VMEM((_C, 128), jnp.float32),
            pltpu.VMEM_SHARED((_NPAD, 128), jnp.float32),
            pltpu.SemaphoreType.DMA,
            pltpu.SemaphoreType.DMA,
        ],
    )
    def agg(ga_hbm, gb_hbm, src_hbm, dst_hbm, z_hbm, out_hbm,
            src_blk, dst_blk, rows0, rows1, acc, gsem, ssem):
        sid = lax.axis_index("s")
        cid = lax.axis_index("c")
        rows = (rows0, rows1)

        pltpu.sync_copy(z_hbm, acc.at[pl.ds(sid * zr, zr)])
        plsc.subcore_barrier()

        def run(gt, ot, row_off):
            @pl.loop(0, tile_rows // KB)
            def _(b):
                r0 = row_off + sid * tile_rows + b * KB
                pltpu.sync_copy(src_hbm.at[pl.ds(r0, KB)], src_blk)
                pltpu.sync_copy(dst_hbm.at[pl.ds(r0, KB)], dst_blk)
                pltpu.async_copy(gt.at[src_blk.at[0]], rows[0], gsem)
                for k in range(KB):
                    cur = rows[k % 2]
                    if k >= 1:
                        pltpu.make_async_copy(
                            rows[(k - 1) % 2],
                            acc.at[dst_blk.at[k - 1]], ssem).wait()
                    if k + 1 < KB:
                        pltpu.async_copy(gt.at[src_blk.at[k + 1]],
                                         rows[(k + 1) % 2], gsem)
                    pltpu.make_async_copy(gt.at[src_blk.at[k]],
                                          cur, gsem).wait()
                    pltpu.async_copy(cur, acc.at[dst_blk.at[k]], ssem,
                                     add=True)
                pltpu.make_async_copy(rows[(KB - 1) % 2],
                                      acc.at[dst_blk.at[KB - 1]],
                                      ssem).wait()

            plsc.subcore_barrier()
            pltpu.sync_copy(acc.at[pl.ds(sid * zr, zr)],
                            ot.at[pl.ds(sid * zr, zr)])

        @pl.when(cid == 0)
        def _():
            run(ga_hbm, out_hbm.at[0], 0)

        @pl.when(cid == 1)
        def _():
            run(gb_hbm, out_hbm.at[1], _EP // 256 if split_edges else 0)

    return agg


_agg_edge = _make_agg(True)
_agg_feat = _make_agg(False)


def _dinv_body(dp_ref, dinv_ref):
    deg = jnp.sum(dp_ref[...], axis=0, keepdims=True) + 1.0
    dinv_ref[...] = lax.rsqrt(deg)


def _scale_body(x_ref, dinv_ref, o_ref):
    o_ref[...] = x_ref[...] * dinv_ref[...]


def _stage1_body(sa_ref, sb_ref, g_ref, dinv_ref, w_ref, b_ref,
                 oa_ref, ob_ref):
    h = (sa_ref[0] + sb_ref[0] + g_ref[...]) * dinv_ref[...]
    y = jnp.maximum(jnp.dot(h, w_ref[...],
                            preferred_element_type=jnp.float32)
                    + b_ref[...], 0.0)
    g1 = y * dinv_ref[...]
    oa_ref[...] = g1[:, :128]
    ob_ref[...] = g1[:, 128:]


def _stage2_body(sa_ref, sb_ref, ga_ref, gb_ref, dinv_ref,
                 w2_ref, b2_ref, w3_ref, o_ref):
    h = jnp.concatenate([sa_ref[0] + ga_ref[...],
                         sb_ref[0] + gb_ref[...]], axis=1) * dinv_ref[...]
    y = jnp.maximum(jnp.dot(h, w2_ref[...],
                            preferred_element_type=jnp.float32)
                    + b2_ref[...], 0.0)
    z = jnp.dot(y, w3_ref[...], preferred_element_type=jnp.float32)
    o_ref[...] = z * dinv_ref[...]


def _stage3_body(sa_ref, sb_ref, g_ref, dinv_ref, b_ref, o_ref):
    o_ref[...] = ((sa_ref[0] + sb_ref[0] + g_ref[...]) * dinv_ref[...]
                  + b_ref[...])


_BN = 2000


def _row_spec(f):
    return pl.BlockSpec((_BN, f), lambda i: (i, 0))


def _full_spec(shape):
    return pl.BlockSpec(shape, lambda i: (0, 0))


def _part_spec(c):
    return pl.BlockSpec((1, _BN, 128), lambda i, c=c: (c, i, 0))


def kernel(x, edge_index, W1, b1, W2, b2, W3, b3):
    src = edge_index[0]
    dst = edge_index[1]
    pad = _EP - E
    pad_i = jnp.arange(pad, dtype=jnp.int32)
    src_p = jnp.concatenate([src, pad_i % N]).reshape(_EP // _C, _C)
    dst_p = jnp.concatenate([dst, N + (pad_i % (_NPAD - N))]
                            ).reshape(_EP // _C, _C)

    zeros128 = jnp.zeros((_NPAD // 16, 128), jnp.float32)

    dp = _deg_kernel(dst_p)
    dinv_row = pl.pallas_call(
        _dinv_body,
        out_shape=jax.ShapeDtypeStruct((1, _NPAD), jnp.float32),
    )(dp)
    dinv = dinv_row.reshape(_NPAD, 1)[:N]

    g0 = pl.pallas_call(
        _scale_body,
        grid=(N // _BN,),
        in_specs=[_row_spec(F_IN), _row_spec(1)],
        out_specs=_row_spec(F_IN),
        out_shape=jax.ShapeDtypeStruct((N, F_IN), jnp.float32),
    )(x, dinv)
    s0 = _agg_edge(g0, g0, src_p, dst_p, zeros128)

    ga1, gb1 = pl.pallas_call(
        _stage1_body,
        grid=(N // _BN,),
        in_specs=[_part_spec(0), _part_spec(1), _row_spec(F_IN),
                  _row_spec(1), _full_spec((F_IN, HID)), _full_spec((1, HID))],
        out_specs=[_row_spec(128), _row_spec(128)],
        out_shape=[jax.ShapeDtypeStruct((N, 128), jnp.float32),
                   jax.ShapeDtypeStruct((N, 128), jnp.float32)],
    )(s0, s0, g0, dinv, W1, b1.reshape(1, HID))

    s1 = _agg_feat(ga1, gb1, src_p, dst_p, zeros128)

    g2 = pl.pallas_call(
        _stage2_body,
        grid=(N // _BN,),
        in_specs=[_part_spec(0), _part_spec(1), _row_spec(128),
                  _row_spec(128), _row_spec(1),
                  _full_spec((HID, HID)), _full_spec((1, HID)),
                  _full_spec((HID, F_IN))],
        out_specs=_row_spec(F_IN),
        out_shape=jax.ShapeDtypeStruct((N, F_IN), jnp.float32),
    )(s1, s1, ga1, gb1, dinv, W2, b2.reshape(1, HID), W3)

    s2 = _agg_edge(g2, g2, src_p, dst_p, zeros128)

    out = pl.pallas_call(
        _stage3_body,
        grid=(N // _BN,),
        in_specs=[_part_spec(0), _part_spec(1), _row_spec(F_IN),
                  _row_spec(1), _full_spec((1, F_IN))],
        out_specs=_row_spec(F_IN),
        out_shape=jax.ShapeDtypeStruct((N, F_IN), jnp.float32),
    )(s2, s2, g2, dinv, b3.reshape(1, F_IN))
    return out
